# baseline (speedup 1.0000x reference)
import functools

import jax
import jax.numpy as jnp
from jax import lax
from jax.experimental import pallas as pl
from jax.experimental.pallas import tpu as pltpu
from jax.experimental.pallas import tpu_sc as plsc

_B = 16384
_NS, _L = 16, 16
_CHUNK = _B // _NS
_NVEC = _CHUNK // _L

_mesh = plsc.VectorSubcoreMesh(
    core_axis_name="c", subcore_axis_name="s", num_cores=1)


@functools.partial(
    pl.kernel,
    mesh=_mesh,
    out_type=jax.ShapeDtypeStruct((_B,), jnp.float32),
    scratch_types=[
        pltpu.VMEM((_CHUNK,), jnp.float32),
        pltpu.VMEM((_CHUNK,), jnp.float32),
        pltpu.VMEM((_L,), jnp.float32),
        pltpu.SemaphoreType.DMA,
        pltpu.SemaphoreType.DMA,
    ],
)
def _sc_sample(x_hbm, wb_hbm, out_hbm, x_v, o_v, wb_v, sem_x, sem_wb):
    sid = lax.axis_index("s")
    base = sid * _CHUNK
    cp_x = pltpu.async_copy(x_hbm.at[pl.ds(base, _CHUNK)], x_v, sem_x)
    cp_wb = pltpu.async_copy(wb_hbm, wb_v, sem_wb)
    lane = lax.iota(jnp.int32, _L)
    pop = jnp.where(lane < 2, 0.0, jnp.where(lane < 4, 1.0, 0.0))
    cp_wb.wait()
    wb = wb_v[...]
    w = wb[0]
    b = wb[1]
    cp_x.wait()
    for i in range(_NVEC):
        sl = pl.ds(i * _L, _L)
        z = x_v[sl] * w + b
        p0 = 1.0 / (1.0 + jnp.exp(-z))
        p1 = 1.0 - p0
        h = (base + i * _L) + lane
        h = h * jnp.int32(-1640531527)
        h = h ^ (lax.shift_right_logical(h, 15))
        h = h * jnp.int32(-2048144789)
        h = h ^ (lax.shift_right_logical(h, 13))
        u = (h & jnp.int32(0x7FFFFF)).astype(jnp.float32) * (1.0 / 8388608.0)
        u = jnp.clip(u, 1e-7, 1.0 - 1e-7)
        take1 = (p1 + 1e-12) * (1.0 - u) > (p0 + 1e-12) * u
        idx = jnp.where(take1, 1, 0).astype(jnp.int32)
        v = lax.gather(
            pop, idx[:, None],
            lax.GatherDimensionNumbers(
                offset_dims=(), collapsed_slice_dims=(0,),
                start_index_map=(0,)),
            slice_sizes=(1,),
            mode=lax.GatherScatterMode.PROMISE_IN_BOUNDS)
        v = jnp.where(v <= 0.5, 10.0, v)
        v = jnp.where(v > 0.5, 1.0, v)
        o_v[sl] = v
    pltpu.sync_copy(o_v, out_hbm.at[pl.ds(base, _CHUNK)])


def kernel(input, W, b):
    x = input.reshape(_B)
    wb = jnp.concatenate([W.reshape(1), b.reshape(1),
                          jnp.zeros((_L - 2,), jnp.float32)])
    out = _sc_sample(x, wb)
    return out.reshape(_B, 1)


# speedup vs baseline: 1.0026x; 1.0026x over previous
import functools

import jax
import jax.numpy as jnp
from jax import lax
from jax.experimental import pallas as pl
from jax.experimental.pallas import tpu as pltpu
from jax.experimental.pallas import tpu_sc as plsc

_B = 16384
_NS, _L = 16, 16
_CHUNK = _B // _NS
_NVEC = _CHUNK // _L

_mesh = plsc.VectorSubcoreMesh(
    core_axis_name="c", subcore_axis_name="s", num_cores=1)


@functools.partial(
    pl.kernel,
    mesh=_mesh,
    out_type=jax.ShapeDtypeStruct((_B,), jnp.float32),
    scratch_types=[
        pltpu.VMEM((_CHUNK,), jnp.float32),
        pltpu.VMEM((_CHUNK,), jnp.float32),
        pltpu.VMEM((_L,), jnp.float32),
        pltpu.SemaphoreType.DMA,
        pltpu.SemaphoreType.DMA,
        pltpu.SemaphoreType.DMA,
        pltpu.SemaphoreType.DMA,
    ],
)
def _sc_sample(x_hbm, wb_hbm, out_hbm, x_v, o_v, wb_v,
               sem_x0, sem_x1, sem_wb, sem_o):
    sid = lax.axis_index("s")
    base = sid * _CHUNK
    half = _CHUNK // 2
    cp_x0 = pltpu.async_copy(x_hbm.at[pl.ds(base, half)],
                             x_v.at[pl.ds(0, half)], sem_x0)
    cp_x1 = pltpu.async_copy(x_hbm.at[pl.ds(base + half, half)],
                             x_v.at[pl.ds(half, half)], sem_x1)
    cp_wb = pltpu.async_copy(wb_hbm, wb_v, sem_wb)
    lane = lax.iota(jnp.int32, _L)
    pop = jnp.where(lane < 2, 0.0, jnp.where(lane < 4, 1.0, 0.0))
    cp_wb.wait()
    wb = wb_v[...]
    w = wb[0]
    b = wb[1]
    cp_x0.wait()
    cp_o0 = None
    for i in range(_NVEC):
        if i == _NVEC // 2:
            cp_o0 = pltpu.async_copy(o_v.at[pl.ds(0, half)],
                                     out_hbm.at[pl.ds(base, half)], sem_o)
            cp_x1.wait()
        sl = pl.ds(i * _L, _L)
        z = x_v[sl] * w + b
        p0 = 1.0 / (1.0 + jnp.exp(-z))
        p1 = 1.0 - p0
        h = (base + i * _L) + lane
        h = h * jnp.int32(-1640531527)
        h = h ^ (lax.shift_right_logical(h, 15))
        h = h * jnp.int32(-2048144789)
        h = h ^ (lax.shift_right_logical(h, 13))
        u = (h & jnp.int32(0x7FFFFF)).astype(jnp.float32) * (1.0 / 8388608.0)
        u = jnp.clip(u, 1e-7, 1.0 - 1e-7)
        take1 = (p1 + 1e-12) * (1.0 - u) > (p0 + 1e-12) * u
        idx = jnp.where(take1, 1, 0).astype(jnp.int32)
        v = lax.gather(
            pop, idx[:, None],
            lax.GatherDimensionNumbers(
                offset_dims=(), collapsed_slice_dims=(0,),
                start_index_map=(0,)),
            slice_sizes=(1,),
            mode=lax.GatherScatterMode.PROMISE_IN_BOUNDS)
        v = jnp.where(v <= 0.5, 10.0, v)
        v = jnp.where(v > 0.5, 1.0, v)
        o_v[sl] = v
    cp_o1 = pltpu.async_copy(o_v.at[pl.ds(half, half)],
                             out_hbm.at[pl.ds(base + half, half)], sem_o)
    cp_o0.wait()
    cp_o1.wait()


def kernel(input, W, b):
    x = input.reshape(_B)
    wb = jnp.concatenate([W.reshape(1), b.reshape(1),
                          jnp.zeros((_L - 2,), jnp.float32)])
    out = _sc_sample(x, wb)
    return out.reshape(_B, 1)
